# baseline (speedup 1.0000x reference)
import functools

import jax
import jax.numpy as jnp
from jax import lax
from jax.experimental import pallas as pl
from jax.experimental.pallas import tpu as pltpu
from jax.experimental.pallas import tpu_sc as plsc

_LN2 = 0.6931471805599453
_TN = 200
_S = 5


def _ssp(x):
    m = jnp.maximum(x, 0.0)
    return m + jnp.log(jnp.exp(x - m) + jnp.exp(-m)) - _LN2


def _dot(a, b):
    return jax.lax.dot_general(a, b, (((a.ndim - 1,), (0,)), ((), ())),
                               preferred_element_type=jnp.float32)


def _sc_gather(table, idx):
    V, Fd = table.shape
    B = idx.shape[0]
    try:
        info = plsc.get_sparse_core_info()
        nc, ns = info.num_cores, info.num_subcores
    except Exception:
        nc, ns = 2, 16
    nw = nc * ns
    per = B // nw
    assert per * nw == B and per % 8 == 0
    ch = 128
    nbuf = 4
    full = per // ch
    groups = full // nbuf
    rest = full - groups * nbuf
    tail = per - full * ch
    mesh = plsc.VectorSubcoreMesh(core_axis_name="c", subcore_axis_name="s")

    @functools.partial(
        pl.kernel, mesh=mesh,
        out_type=jax.ShapeDtypeStruct((B, Fd), jnp.float32),
        scratch_types=[
            pltpu.VMEM((per,), jnp.int32),
            pltpu.VMEM((nbuf, ch, Fd), jnp.float32),
            pltpu.SemaphoreType.DMA,
            pltpu.SemaphoreType.DMA,
        ],
    )
    def gather(table_hbm, idx_hbm, out_hbm, idx_v, rows_v, sem_g, sem_w):
        wid = lax.axis_index("s") * nc + lax.axis_index("c")
        base = wid * per
        pltpu.sync_copy(idx_hbm.at[pl.ds(base, per)], idx_v)

        def group(g, carry):
            off0 = g * nbuf * ch

            @pl.when(g > 0)
            def _():
                for b in range(nbuf):
                    pltpu.make_async_copy(
                        rows_v.at[b],
                        out_hbm.at[pl.ds(base + off0 + b * ch, ch)],
                        sem_w).wait()

            for b in range(nbuf):
                pltpu.async_copy(
                    table_hbm.at[idx_v.at[pl.ds(off0 + b * ch, ch)]],
                    rows_v.at[b], sem_g)

            for b in range(nbuf):
                pltpu.make_async_copy(
                    table_hbm.at[idx_v.at[pl.ds(off0 + b * ch, ch)]],
                    rows_v.at[b], sem_g).wait()
                pltpu.async_copy(
                    rows_v.at[b],
                    out_hbm.at[pl.ds(base + off0 + b * ch, ch)], sem_w)
            return carry

        lax.fori_loop(0, groups, group, 0)
        for b in range(nbuf):
            pltpu.make_async_copy(
                rows_v.at[b], out_hbm.at[pl.ds(base, ch)], sem_w).wait()
        for r in range(rest):
            off = (groups * nbuf + r) * ch
            pltpu.async_copy(table_hbm.at[idx_v.at[pl.ds(off, ch)]],
                             rows_v.at[0], sem_g).wait()
            pltpu.sync_copy(rows_v.at[0], out_hbm.at[pl.ds(base + off, ch)])
        if tail:
            off = full * ch
            pltpu.async_copy(
                table_hbm.at[idx_v.at[pl.ds(off, tail)]],
                rows_v.at[0].at[pl.ds(0, tail)], sem_g).wait()
            pltpu.sync_copy(rows_v.at[0].at[pl.ds(0, tail)],
                            out_hbm.at[pl.ds(base + off, tail)])

    return gather(table, idx)


def _embed_call(Zf, emb, w_in, b_in):
    N = Zf.shape[0]
    A, Fd = emb.shape
    grid = N // _TN

    def body(z_ref, emb_ref, wi_ref, bi_ref, x_ref, v_ref):
        ar = lax.broadcasted_iota(jnp.int32, (_TN, A), 1)
        onehot = (ar == z_ref[...]).astype(jnp.float32)
        x = _dot(onehot, emb_ref[...])
        x_ref[...] = x
        v_ref[...] = _dot(x, wi_ref[...]) + bi_ref[...]

    return pl.pallas_call(
        body,
        grid=(grid,),
        in_specs=[
            pl.BlockSpec((_TN, 1), lambda i: (i, 0)),
            pl.BlockSpec((A, Fd), lambda i: (0, 0)),
            pl.BlockSpec((Fd, Fd), lambda i: (0, 0)),
            pl.BlockSpec((1, Fd), lambda i: (0, 0)),
        ],
        out_specs=[
            pl.BlockSpec((_TN, Fd), lambda i: (i, 0)),
            pl.BlockSpec((_TN, Fd), lambda i: (i, 0)),
        ],
        out_shape=[
            jax.ShapeDtypeStruct((N, Fd), jnp.float32),
            jax.ShapeDtypeStruct((N, Fd), jnp.float32),
        ],
    )(Zf, emb, w_in, b_in)


def _interaction_call(x, x_off, vj, rbf3, rbf_off, blk, nxt, K, n_atoms):
    Fd = x.shape[-1]
    R = rbf3.shape[-1]
    grid = n_atoms // _TN
    rows = _TN * K

    def body(*refs):
        (rbf_ref, vj_ref, x_ref, fw1, fb1, fw2, fb2, w1, b1, w2, b2) = refs[:11]
        rest = refs[11:]
        u = _ssp(_dot(rbf_ref[...].reshape(rows, R), fw1[...]) + fb1[...])
        w = _dot(u, fw2[...]) + fb2[...]
        p = w * vj_ref[...]
        y = p.reshape(_TN, K, Fd).sum(axis=1)
        y = _ssp(_dot(y, w1[...]) + b1[...])
        y = _dot(y, w2[...]) + b2[...]
        xo = x_ref[...] + y
        if nxt is not None:
            wi, bi, xo_ref, vn_ref = rest
            xo_ref[...] = xo
            vn_ref[...] = _dot(xo, wi[...]) + bi[...]
        else:
            (xo_ref,) = rest
            xo_ref[...] = xo

    wspec = lambda s: pl.BlockSpec(s, lambda i: (0, 0))
    in_specs = [
        pl.BlockSpec((_TN, K, R), lambda i: (i + rbf_off, 0, 0)),
        pl.BlockSpec((rows, Fd), lambda i: (i, 0)),
        pl.BlockSpec((_TN, Fd), lambda i: (i + x_off, 0)),
        wspec((R, Fd)), wspec((1, Fd)), wspec((Fd, Fd)), wspec((1, Fd)),
        wspec((Fd, Fd)), wspec((1, Fd)), wspec((Fd, Fd)), wspec((1, Fd)),
    ]
    args = [rbf3, vj, x,
            blk["fw1"], blk["fb1"].reshape(1, Fd),
            blk["fw2"], blk["fb2"].reshape(1, Fd),
            blk["w1"], blk["b1"].reshape(1, Fd),
            blk["w2"], blk["b2"].reshape(1, Fd)]
    out_specs = [pl.BlockSpec((_TN, Fd), lambda i: (i, 0))]
    out_shape = [jax.ShapeDtypeStruct((n_atoms, Fd), jnp.float32)]
    if nxt is not None:
        in_specs += [wspec((Fd, Fd)), wspec((1, Fd))]
        args += [nxt["w_in"], nxt["b_in"].reshape(1, Fd)]
        out_specs.append(pl.BlockSpec((_TN, Fd), lambda i: (i, 0)))
        out_shape.append(jax.ShapeDtypeStruct((n_atoms, Fd), jnp.float32))

    out = pl.pallas_call(
        body, grid=(grid,), in_specs=in_specs, out_specs=out_specs,
        out_shape=out_shape,
    )(*args)
    return (out[0], out[1]) if nxt is not None else (out[0], None)


def _readout_call(x, ro):
    N, Fd = x.shape
    H = ro["rw1"].shape[1]
    grid = N // _TN

    def body(x_ref, rw1, rb1, rw2, rb2, out_ref, acc_ref):
        i = pl.program_id(0)

        @pl.when(i == 0)
        def _():
            acc_ref[...] = jnp.zeros_like(acc_ref)

        h = _ssp(_dot(x_ref[...], rw1[...]) + rb1[...])
        acc_ref[...] += jnp.sum(h, axis=0, keepdims=True)

        @pl.when(i == grid - 1)
        def _():
            out_ref[...] = _dot(acc_ref[...], rw2[...]) + N * rb2[...]

    wspec = lambda s: pl.BlockSpec(s, lambda i: (0, 0))
    return pl.pallas_call(
        body,
        grid=(grid,),
        in_specs=[
            pl.BlockSpec((_TN, Fd), lambda i: (i, 0)),
            wspec((Fd, H)), wspec((1, H)), wspec((H, 1)), wspec((1, 1)),
        ],
        out_specs=pl.BlockSpec((1, 1), lambda i: (0, 0)),
        out_shape=jax.ShapeDtypeStruct((1, 1), jnp.float32),
        scratch_shapes=[pltpu.VMEM((1, H), jnp.float32)],
        compiler_params=pltpu.CompilerParams(
            dimension_semantics=("arbitrary",)),
    )(x, ro["rw1"], ro["rb1"].reshape(1, H), ro["rw2"],
      ro["rb2"].reshape(1, 1))


def kernel(Z, rbf, neighbors, params):
    emb = params["embedding"]
    blocks = params["blocks"]
    ro = params["readout"]
    N, K = neighbors.shape
    R = rbf.shape[-1]
    Fd = emb.shape[1]

    nbr = neighbors.reshape(N * K).astype(jnp.int32)
    Zf = Z.astype(jnp.int32).reshape(N, 1)

    x, v = _embed_call(Zf, emb, blocks[0]["w_in"],
                       blocks[0]["b_in"].reshape(1, Fd))

    ps = N // _S
    ptiles = ps // _TN
    nbr_parts = [nbr[p * ps * K:(p + 1) * ps * K] for p in range(_S)]
    xs = None
    for t in range(len(blocks)):
        nxt = blocks[t + 1] if t + 1 < len(blocks) else None
        new_xs, new_vs = [], []
        for p in range(_S):
            vj_p = _sc_gather(v, nbr_parts[p])
            if xs is None:
                xa, xo = x, p * ptiles
            else:
                xa, xo = xs[p], 0
            xp, vp = _interaction_call(xa, xo, vj_p, rbf, p * ptiles,
                                       blocks[t], nxt, K, ps)
            new_xs.append(xp)
            new_vs.append(vp)
        xs = new_xs
        if nxt is not None:
            v = jnp.concatenate(new_vs, axis=0)
    e = _readout_call(jnp.concatenate(xs, axis=0), ro)
    return e.reshape(())


# speedup vs baseline: 1.0538x; 1.0538x over previous
import functools

import jax
import jax.numpy as jnp
from jax import lax
from jax.experimental import pallas as pl
from jax.experimental.pallas import tpu as pltpu
from jax.experimental.pallas import tpu_sc as plsc

_LN2 = 0.6931471805599453
_TN = 200
_S = 2


def _ssp(x):
    m = jnp.maximum(x, 0.0)
    return m + jnp.log(jnp.exp(x - m) + jnp.exp(-m)) - _LN2


def _dot(a, b):
    return jax.lax.dot_general(a, b, (((a.ndim - 1,), (0,)), ((), ())),
                               preferred_element_type=jnp.float32)


def _sc_gather(table, idx, part_off, B):
    V, Fd = table.shape
    try:
        info = plsc.get_sparse_core_info()
        nc, ns = info.num_cores, info.num_subcores
    except Exception:
        nc, ns = 2, 16
    nw = nc * ns
    per = B // nw
    assert per * nw == B and per % 8 == 0 and part_off % 8 == 0
    ch = 128
    nbuf = 4
    full = per // ch
    groups = full // nbuf
    rest = full - groups * nbuf
    tail = per - full * ch
    mesh = plsc.VectorSubcoreMesh(core_axis_name="c", subcore_axis_name="s")

    @functools.partial(
        pl.kernel, mesh=mesh,
        out_type=jax.ShapeDtypeStruct((B, Fd), jnp.float32),
        scratch_types=[
            pltpu.VMEM((per,), jnp.int32),
            pltpu.VMEM((nbuf, ch, Fd), jnp.float32),
            pltpu.SemaphoreType.DMA,
            pltpu.SemaphoreType.DMA,
        ],
    )
    def gather(table_hbm, idx_hbm, out_hbm, idx_v, rows_v, sem_g, sem_w):
        wid = lax.axis_index("s") * nc + lax.axis_index("c")
        base = wid * per
        pltpu.sync_copy(idx_hbm.at[pl.ds(part_off + base, per)], idx_v)

        def group(g, carry):
            off0 = g * nbuf * ch

            @pl.when(g > 0)
            def _():
                for b in range(nbuf):
                    pltpu.make_async_copy(
                        rows_v.at[b],
                        out_hbm.at[pl.ds(base + off0 + b * ch, ch)],
                        sem_w).wait()

            for b in range(nbuf):
                pltpu.async_copy(
                    table_hbm.at[idx_v.at[pl.ds(off0 + b * ch, ch)]],
                    rows_v.at[b], sem_g)

            for b in range(nbuf):
                pltpu.make_async_copy(
                    table_hbm.at[idx_v.at[pl.ds(off0 + b * ch, ch)]],
                    rows_v.at[b], sem_g).wait()
                pltpu.async_copy(
                    rows_v.at[b],
                    out_hbm.at[pl.ds(base + off0 + b * ch, ch)], sem_w)
            return carry

        lax.fori_loop(0, groups, group, 0)
        for b in range(nbuf):
            pltpu.make_async_copy(
                rows_v.at[b], out_hbm.at[pl.ds(base, ch)], sem_w).wait()
        for r in range(rest):
            off = (groups * nbuf + r) * ch
            pltpu.async_copy(table_hbm.at[idx_v.at[pl.ds(off, ch)]],
                             rows_v.at[0], sem_g).wait()
            pltpu.sync_copy(rows_v.at[0], out_hbm.at[pl.ds(base + off, ch)])
        if tail:
            off = full * ch
            pltpu.async_copy(
                table_hbm.at[idx_v.at[pl.ds(off, tail)]],
                rows_v.at[0].at[pl.ds(0, tail)], sem_g).wait()
            pltpu.sync_copy(rows_v.at[0].at[pl.ds(0, tail)],
                            out_hbm.at[pl.ds(base + off, tail)])

    return gather(table, idx)


def _embed_call(Zi, emb, w_in, b_in):
    N = Zi.shape[0]
    A, Fd = emb.shape
    grid = N // _TN

    def body(z_ref, emb_ref, wi_ref, bi_ref, x_ref, v_ref):
        ar = lax.broadcasted_iota(jnp.int32, (_TN, A), 1)
        onehot = (ar == z_ref[...]).astype(jnp.float32)
        x = _dot(onehot, emb_ref[...])
        x_ref[...] = x
        v_ref[...] = _dot(x, wi_ref[...]) + bi_ref[...]

    return pl.pallas_call(
        body,
        grid=(grid,),
        in_specs=[
            pl.BlockSpec((_TN, 1), lambda i: (i, 0)),
            pl.BlockSpec((A, Fd), lambda i: (0, 0)),
            pl.BlockSpec((Fd, Fd), lambda i: (0, 0)),
            pl.BlockSpec((1, Fd), lambda i: (0, 0)),
        ],
        out_specs=[
            pl.BlockSpec((_TN, Fd), lambda i: (i, 0)),
            pl.BlockSpec((_TN, Fd), lambda i: (i, 0)),
        ],
        out_shape=[
            jax.ShapeDtypeStruct((N, Fd), jnp.float32),
            jax.ShapeDtypeStruct((N, Fd), jnp.float32),
        ],
    )(Zi, emb, w_in, b_in)


def _interaction_mid(x, x_off, vj, rbf3, p_off, blk, nxt, K, ps, vbuf):
    Fd = x.shape[-1]
    R = rbf3.shape[-1]
    Nfull = vbuf.shape[0]
    grid = ps // _TN
    rows = _TN * K

    def body(rbf_ref, vj_ref, x_ref, fw1, fb1, fw2, fb2, w1, b1, w2, b2,
             wi, bi, vb_ref, xo_ref, vn_ref):
        u = _ssp(_dot(rbf_ref[...].reshape(rows, R), fw1[...]) + fb1[...])
        w = _dot(u, fw2[...]) + fb2[...]
        p = w * vj_ref[...]
        y = p.reshape(_TN, K, Fd).sum(axis=1)
        y = _ssp(_dot(y, w1[...]) + b1[...])
        y = _dot(y, w2[...]) + b2[...]
        xo = x_ref[...] + y
        xo_ref[...] = xo
        vn_ref[...] = _dot(xo, wi[...]) + bi[...]

    wspec = lambda s: pl.BlockSpec(s, lambda i: (0, 0))
    return pl.pallas_call(
        body,
        grid=(grid,),
        in_specs=[
            pl.BlockSpec((_TN, K, R), lambda i: (i + p_off, 0, 0)),
            pl.BlockSpec((rows, Fd), lambda i: (i, 0)),
            pl.BlockSpec((_TN, Fd), lambda i: (i + x_off, 0)),
            wspec((R, Fd)), wspec((1, Fd)), wspec((Fd, Fd)), wspec((1, Fd)),
            wspec((Fd, Fd)), wspec((1, Fd)), wspec((Fd, Fd)), wspec((1, Fd)),
            wspec((Fd, Fd)), wspec((1, Fd)),
            pl.BlockSpec(memory_space=pl.ANY),
        ],
        out_specs=[
            pl.BlockSpec((_TN, Fd), lambda i: (i, 0)),
            pl.BlockSpec((_TN, Fd), lambda i: (i + p_off, 0)),
        ],
        out_shape=[
            jax.ShapeDtypeStruct((ps, Fd), jnp.float32),
            jax.ShapeDtypeStruct((Nfull, Fd), jnp.float32),
        ],
        input_output_aliases={13: 1},
    )(rbf3, vj, x, blk["fw1"], blk["fb1"].reshape(1, Fd),
      blk["fw2"], blk["fb2"].reshape(1, Fd),
      blk["w1"], blk["b1"].reshape(1, Fd),
      blk["w2"], blk["b2"].reshape(1, Fd),
      nxt["w_in"], nxt["b_in"].reshape(1, Fd), vbuf)


def _interaction_last(x, vj, rbf3, p_off, blk, ro, K, ps, n_total):
    Fd = x.shape[-1]
    R = rbf3.shape[-1]
    H = ro["rw1"].shape[1]
    grid = ps // _TN
    rows = _TN * K

    def body(rbf_ref, vj_ref, x_ref, fw1, fb1, fw2, fb2, w1, b1, w2, b2,
             rw1, rb1, hs_ref):
        u = _ssp(_dot(rbf_ref[...].reshape(rows, R), fw1[...]) + fb1[...])
        w = _dot(u, fw2[...]) + fb2[...]
        p = w * vj_ref[...]
        y = p.reshape(_TN, K, Fd).sum(axis=1)
        y = _ssp(_dot(y, w1[...]) + b1[...])
        y = _dot(y, w2[...]) + b2[...]
        xo = x_ref[...] + y
        h = _ssp(_dot(xo, rw1[...]) + rb1[...])
        i = pl.program_id(0)

        @pl.when(i == 0)
        def _():
            hs_ref[...] = jnp.zeros_like(hs_ref)

        hs_ref[...] += jnp.sum(h, axis=0, keepdims=True)

    wspec = lambda s: pl.BlockSpec(s, lambda i: (0, 0))
    return pl.pallas_call(
        body,
        grid=(grid,),
        in_specs=[
            pl.BlockSpec((_TN, K, R), lambda i: (i + p_off, 0, 0)),
            pl.BlockSpec((rows, Fd), lambda i: (i, 0)),
            pl.BlockSpec((_TN, Fd), lambda i: (i, 0)),
            wspec((R, Fd)), wspec((1, Fd)), wspec((Fd, Fd)), wspec((1, Fd)),
            wspec((Fd, Fd)), wspec((1, Fd)), wspec((Fd, Fd)), wspec((1, Fd)),
            wspec((Fd, H)), wspec((1, H)),
        ],
        out_specs=pl.BlockSpec((1, H), lambda i: (0, 0)),
        out_shape=jax.ShapeDtypeStruct((1, H), jnp.float32),
        compiler_params=pltpu.CompilerParams(
            dimension_semantics=("arbitrary",)),
    )(rbf3, vj, x, blk["fw1"], blk["fb1"].reshape(1, Fd),
      blk["fw2"], blk["fb2"].reshape(1, Fd),
      blk["w1"], blk["b1"].reshape(1, Fd),
      blk["w2"], blk["b2"].reshape(1, Fd),
      ro["rw1"], ro["rb1"].reshape(1, H))


def _combine_call(hs, rw2, rb2, n_atoms):
    S, H = hs.shape

    def body(hs_ref, rw2_ref, rb2_ref, out_ref):
        tot = jnp.sum(hs_ref[...], axis=0, keepdims=True)
        out_ref[...] = _dot(tot, rw2_ref[...]) + n_atoms * rb2_ref[...]

    wspec = lambda s: pl.BlockSpec(s, lambda i: (0, 0))
    return pl.pallas_call(
        body,
        grid=(1,),
        in_specs=[wspec((S, H)), wspec((H, 1)), wspec((1, 1))],
        out_specs=pl.BlockSpec((1, 1), lambda i: (0, 0)),
        out_shape=jax.ShapeDtypeStruct((1, 1), jnp.float32),
    )(hs, rw2, rb2)


def kernel(Z, rbf, neighbors, params):
    emb = params["embedding"]
    blocks = params["blocks"]
    ro = params["readout"]
    N, K = neighbors.shape
    R = rbf.shape[-1]
    Fd = emb.shape[1]
    T = len(blocks)

    nbr = neighbors.reshape(N * K).astype(jnp.int32)
    Zi = Z.astype(jnp.int32).reshape(N, 1)

    ps = N // _S
    ptiles = ps // _TN

    x, v = _embed_call(Zi, emb, blocks[0]["w_in"],
                       blocks[0]["b_in"].reshape(1, Fd))
    xs = None
    for t in range(T - 1):
        vbuf = jnp.zeros((N, Fd), jnp.float32)
        new_xs = []
        for p in range(_S):
            vj_p = _sc_gather(v, nbr, p * ps * K, ps * K)
            if xs is None:
                xa, xo = x, p * ptiles
            else:
                xa, xo = xs[p], 0
            xp, vbuf = _interaction_mid(xa, xo, vj_p, rbf, p * ptiles,
                                        blocks[t], blocks[t + 1], K, ps, vbuf)
            new_xs.append(xp)
        xs = new_xs
        v = vbuf

    hs = []
    for p in range(_S):
        vj_p = _sc_gather(v, nbr, p * ps * K, ps * K)
        hs.append(_interaction_last(xs[p], vj_p, rbf, p * ptiles,
                                    blocks[T - 1], ro, K, ps, N))
    e = _combine_call(jnp.concatenate(hs, axis=0), ro["rw2"],
                      ro["rb2"].reshape(1, 1), N)
    return e.reshape(())


# speedup vs baseline: 1.1174x; 1.0604x over previous
import functools

import jax
import jax.numpy as jnp
from jax import lax
from jax.experimental import pallas as pl
from jax.experimental.pallas import tpu as pltpu
from jax.experimental.pallas import tpu_sc as plsc

_LN2 = 0.6931471805599453
_TN = 200


def _ssp(x):
    m = jnp.maximum(x, 0.0)
    return m + jnp.log(jnp.exp(x - m) + jnp.exp(-m)) - _LN2


def _dot(a, b):
    return jax.lax.dot_general(a, b, (((a.ndim - 1,), (0,)), ((), ())),
                               preferred_element_type=jnp.float32)


def _bdot(a, b):
    return _dot(a.astype(jnp.bfloat16), b.astype(jnp.bfloat16))


def _sc_gather(table, idx):
    V, Fd = table.shape
    B = idx.shape[0]
    try:
        info = plsc.get_sparse_core_info()
        nc, ns = info.num_cores, info.num_subcores
    except Exception:
        nc, ns = 2, 16
    nw = nc * ns
    per = B // nw
    assert per * nw == B and per % 8 == 0
    ch = 128
    nbuf = 4
    full = per // ch
    groups = full // nbuf
    rest = full - groups * nbuf
    tail = per - full * ch
    mesh = plsc.VectorSubcoreMesh(core_axis_name="c", subcore_axis_name="s")

    @functools.partial(
        pl.kernel, mesh=mesh,
        out_type=jax.ShapeDtypeStruct((B, Fd), table.dtype),
        scratch_types=[
            pltpu.VMEM((per,), jnp.int32),
            pltpu.VMEM((nbuf, ch, Fd), table.dtype),
            pltpu.SemaphoreType.DMA,
            pltpu.SemaphoreType.DMA,
        ],
    )
    def gather(table_hbm, idx_hbm, out_hbm, idx_v, rows_v, sem_g, sem_w):
        wid = lax.axis_index("s") * nc + lax.axis_index("c")
        base = wid * per
        pltpu.sync_copy(idx_hbm.at[pl.ds(base, per)], idx_v)

        def group(g, carry):
            off0 = g * nbuf * ch

            @pl.when(g > 0)
            def _():
                for b in range(nbuf):
                    pltpu.make_async_copy(
                        rows_v.at[b],
                        out_hbm.at[pl.ds(base + off0 + b * ch, ch)],
                        sem_w).wait()

            for b in range(nbuf):
                pltpu.async_copy(
                    table_hbm.at[idx_v.at[pl.ds(off0 + b * ch, ch)]],
                    rows_v.at[b], sem_g)

            for b in range(nbuf):
                pltpu.make_async_copy(
                    table_hbm.at[idx_v.at[pl.ds(off0 + b * ch, ch)]],
                    rows_v.at[b], sem_g).wait()
                pltpu.async_copy(
                    rows_v.at[b],
                    out_hbm.at[pl.ds(base + off0 + b * ch, ch)], sem_w)
            return carry

        lax.fori_loop(0, groups, group, 0)
        for b in range(nbuf):
            pltpu.make_async_copy(
                rows_v.at[b], out_hbm.at[pl.ds(base, ch)], sem_w).wait()
        for r in range(rest):
            off = (groups * nbuf + r) * ch
            pltpu.async_copy(table_hbm.at[idx_v.at[pl.ds(off, ch)]],
                             rows_v.at[0], sem_g).wait()
            pltpu.sync_copy(rows_v.at[0], out_hbm.at[pl.ds(base + off, ch)])
        if tail:
            off = full * ch
            pltpu.async_copy(
                table_hbm.at[idx_v.at[pl.ds(off, tail)]],
                rows_v.at[0].at[pl.ds(0, tail)], sem_g).wait()
            pltpu.sync_copy(rows_v.at[0].at[pl.ds(0, tail)],
                            out_hbm.at[pl.ds(base + off, tail)])

    return gather(table, idx)


def _embed_call(Zi, emb, w_in, b_in):
    N = Zi.shape[0]
    A, Fd = emb.shape
    grid = N // _TN

    def body(z_ref, emb_ref, wi_ref, bi_ref, x_ref, v_ref):
        ar = lax.broadcasted_iota(jnp.int32, (_TN, A), 1)
        onehot = (ar == z_ref[...]).astype(jnp.float32)
        x = _dot(onehot, emb_ref[...])
        x_ref[...] = x
        v_ref[...] = _dot(x, wi_ref[...]) + bi_ref[...]

    return pl.pallas_call(
        body,
        grid=(grid,),
        in_specs=[
            pl.BlockSpec((_TN, 1), lambda i: (i, 0)),
            pl.BlockSpec((A, Fd), lambda i: (0, 0)),
            pl.BlockSpec((Fd, Fd), lambda i: (0, 0)),
            pl.BlockSpec((1, Fd), lambda i: (0, 0)),
        ],
        out_specs=[
            pl.BlockSpec((_TN, Fd), lambda i: (i, 0)),
            pl.BlockSpec((_TN, Fd), lambda i: (i, 0)),
        ],
        out_shape=[
            jax.ShapeDtypeStruct((N, Fd), jnp.float32),
            jax.ShapeDtypeStruct((N, Fd), jnp.float32),
        ],
    )(Zi, emb, w_in, b_in)


def _filter_conv(rbf_ref, vj_ref, x_ref, fw1, fb1, fw2, fb2, w1, b1, w2, b2,
                 K, Fd, R):
    rows = _TN * K
    u = _ssp(_bdot(rbf_ref[...].reshape(rows, R), fw1[...]) + fb1[...])
    w = _bdot(u, fw2[...]) + fb2[...]
    p = w * vj_ref[...]
    y = p.reshape(_TN, K, Fd).sum(axis=1)
    y = _ssp(_dot(y, w1[...]) + b1[...])
    y = _dot(y, w2[...]) + b2[...]
    return x_ref[...] + y


def _interaction_mid(x, vj, rbf3, blk, nxt, K):
    N, Fd = x.shape
    R = rbf3.shape[-1]
    grid = N // _TN
    rows = _TN * K

    def body(rbf_ref, vj_ref, x_ref, fw1, fb1, fw2, fb2, w1, b1, w2, b2,
             wi, bi, xo_ref, vn_ref):
        xo = _filter_conv(rbf_ref, vj_ref, x_ref, fw1, fb1, fw2, fb2,
                          w1, b1, w2, b2, K, Fd, R)
        xo_ref[...] = xo
        vn_ref[...] = _dot(xo, wi[...]) + bi[...]

    wspec = lambda s: pl.BlockSpec(s, lambda i: (0, 0))
    return pl.pallas_call(
        body,
        grid=(grid,),
        in_specs=[
            pl.BlockSpec((_TN, K, R), lambda i: (i, 0, 0)),
            pl.BlockSpec((rows, Fd), lambda i: (i, 0)),
            pl.BlockSpec((_TN, Fd), lambda i: (i, 0)),
            wspec((R, Fd)), wspec((1, Fd)), wspec((Fd, Fd)), wspec((1, Fd)),
            wspec((Fd, Fd)), wspec((1, Fd)), wspec((Fd, Fd)), wspec((1, Fd)),
            wspec((Fd, Fd)), wspec((1, Fd)),
        ],
        out_specs=[
            pl.BlockSpec((_TN, Fd), lambda i: (i, 0)),
            pl.BlockSpec((_TN, Fd), lambda i: (i, 0)),
        ],
        out_shape=[
            jax.ShapeDtypeStruct((N, Fd), jnp.float32),
            jax.ShapeDtypeStruct((N, Fd), jnp.float32),
        ],
    )(rbf3, vj, x, blk["fw1"], blk["fb1"].reshape(1, Fd),
      blk["fw2"], blk["fb2"].reshape(1, Fd),
      blk["w1"], blk["b1"].reshape(1, Fd),
      blk["w2"], blk["b2"].reshape(1, Fd),
      nxt["w_in"], nxt["b_in"].reshape(1, Fd))


def _interaction_last(x, vj, rbf3, blk, ro, K):
    N, Fd = x.shape
    R = rbf3.shape[-1]
    H = ro["rw1"].shape[1]
    grid = N // _TN
    rows = _TN * K

    def body(rbf_ref, vj_ref, x_ref, fw1, fb1, fw2, fb2, w1, b1, w2, b2,
             rw1, rb1, rw2, rb2, out_ref, acc_ref):
        xo = _filter_conv(rbf_ref, vj_ref, x_ref, fw1, fb1, fw2, fb2,
                          w1, b1, w2, b2, K, Fd, R)
        h = _ssp(_dot(xo, rw1[...]) + rb1[...])
        i = pl.program_id(0)

        @pl.when(i == 0)
        def _():
            acc_ref[...] = jnp.zeros_like(acc_ref)

        acc_ref[...] += jnp.sum(h, axis=0, keepdims=True)

        @pl.when(i == grid - 1)
        def _():
            out_ref[...] = _dot(acc_ref[...], rw2[...]) + N * rb2[...]

    wspec = lambda s: pl.BlockSpec(s, lambda i: (0, 0))
    return pl.pallas_call(
        body,
        grid=(grid,),
        in_specs=[
            pl.BlockSpec((_TN, K, R), lambda i: (i, 0, 0)),
            pl.BlockSpec((rows, Fd), lambda i: (i, 0)),
            pl.BlockSpec((_TN, Fd), lambda i: (i, 0)),
            wspec((R, Fd)), wspec((1, Fd)), wspec((Fd, Fd)), wspec((1, Fd)),
            wspec((Fd, Fd)), wspec((1, Fd)), wspec((Fd, Fd)), wspec((1, Fd)),
            wspec((Fd, H)), wspec((1, H)), wspec((H, 1)), wspec((1, 1)),
        ],
        out_specs=pl.BlockSpec((1, 1), lambda i: (0, 0)),
        out_shape=jax.ShapeDtypeStruct((1, 1), jnp.float32),
        scratch_shapes=[pltpu.VMEM((1, H), jnp.float32)],
        compiler_params=pltpu.CompilerParams(
            dimension_semantics=("arbitrary",)),
    )(rbf3, vj, x, blk["fw1"], blk["fb1"].reshape(1, Fd),
      blk["fw2"], blk["fb2"].reshape(1, Fd),
      blk["w1"], blk["b1"].reshape(1, Fd),
      blk["w2"], blk["b2"].reshape(1, Fd),
      ro["rw1"], ro["rb1"].reshape(1, H), ro["rw2"],
      ro["rb2"].reshape(1, 1))


def kernel(Z, rbf, neighbors, params):
    emb = params["embedding"]
    blocks = params["blocks"]
    ro = params["readout"]
    N, K = neighbors.shape
    R = rbf.shape[-1]
    Fd = emb.shape[1]
    T = len(blocks)

    nbr = neighbors.reshape(N * K).astype(jnp.int32)
    Zi = Z.astype(jnp.int32).reshape(N, 1)
    rbf_b = rbf.astype(jnp.bfloat16)

    x, v = _embed_call(Zi, emb, blocks[0]["w_in"],
                       blocks[0]["b_in"].reshape(1, Fd))
    for t in range(T - 1):
        vj = _sc_gather(v, nbr)
        x, v = _interaction_mid(x, vj, rbf_b, blocks[t], blocks[t + 1], K)
    vj = _sc_gather(v, nbr)
    e = _interaction_last(x, vj, rbf_b, blocks[T - 1], ro, K)
    return e.reshape(())


# speedup vs baseline: 1.1891x; 1.0641x over previous
import functools

import jax
import jax.numpy as jnp
from jax import lax
from jax.experimental import pallas as pl
from jax.experimental.pallas import tpu as pltpu
from jax.experimental.pallas import tpu_sc as plsc

_LN2 = 0.6931471805599453
_TN = 400


def _ssp(x):
    m = jnp.maximum(x, 0.0)
    return m + jnp.log(jnp.exp(x - m) + jnp.exp(-m)) - _LN2


def _dot(a, b):
    return jax.lax.dot_general(a, b, (((a.ndim - 1,), (0,)), ((), ())),
                               preferred_element_type=jnp.float32)


def _bdot(a, b):
    return _dot(a.astype(jnp.bfloat16), b.astype(jnp.bfloat16))


def _sc_gather(table, idx):
    V, Fd = table.shape
    B = idx.shape[0]
    try:
        info = plsc.get_sparse_core_info()
        nc, ns = info.num_cores, info.num_subcores
    except Exception:
        nc, ns = 2, 16
    nw = nc * ns
    per = B // nw
    assert per * nw == B and per % 8 == 0
    ch = 128
    nbuf = 6
    full = per // ch
    groups = full // nbuf
    rest = full - groups * nbuf
    tail = per - full * ch
    mesh = plsc.VectorSubcoreMesh(core_axis_name="c", subcore_axis_name="s")

    @functools.partial(
        pl.kernel, mesh=mesh,
        out_type=jax.ShapeDtypeStruct((B, Fd), table.dtype),
        scratch_types=[
            pltpu.VMEM((per,), jnp.int32),
            pltpu.VMEM((nbuf, ch, Fd), table.dtype),
            pltpu.SemaphoreType.DMA,
            pltpu.SemaphoreType.DMA,
        ],
    )
    def gather(table_hbm, idx_hbm, out_hbm, idx_v, rows_v, sem_g, sem_w):
        wid = lax.axis_index("s") * nc + lax.axis_index("c")
        base = wid * per
        pltpu.sync_copy(idx_hbm.at[pl.ds(base, per)], idx_v)

        def group(g, carry):
            off0 = g * nbuf * ch

            @pl.when(g > 0)
            def _():
                for b in range(nbuf):
                    pltpu.make_async_copy(
                        rows_v.at[b],
                        out_hbm.at[pl.ds(base + off0 + b * ch, ch)],
                        sem_w).wait()

            for b in range(nbuf):
                pltpu.async_copy(
                    table_hbm.at[idx_v.at[pl.ds(off0 + b * ch, ch)]],
                    rows_v.at[b], sem_g)

            for b in range(nbuf):
                pltpu.make_async_copy(
                    table_hbm.at[idx_v.at[pl.ds(off0 + b * ch, ch)]],
                    rows_v.at[b], sem_g).wait()
                pltpu.async_copy(
                    rows_v.at[b],
                    out_hbm.at[pl.ds(base + off0 + b * ch, ch)], sem_w)
            return carry

        lax.fori_loop(0, groups, group, 0)
        for b in range(nbuf):
            pltpu.make_async_copy(
                rows_v.at[b], out_hbm.at[pl.ds(base, ch)], sem_w).wait()
        for r in range(rest):
            off = (groups * nbuf + r) * ch
            pltpu.async_copy(table_hbm.at[idx_v.at[pl.ds(off, ch)]],
                             rows_v.at[0], sem_g).wait()
            pltpu.sync_copy(rows_v.at[0], out_hbm.at[pl.ds(base + off, ch)])
        if tail:
            off = full * ch
            pltpu.async_copy(
                table_hbm.at[idx_v.at[pl.ds(off, tail)]],
                rows_v.at[0].at[pl.ds(0, tail)], sem_g).wait()
            pltpu.sync_copy(rows_v.at[0].at[pl.ds(0, tail)],
                            out_hbm.at[pl.ds(base + off, tail)])

    return gather(table, idx)


def _embed_call(Zi, emb, w_in, b_in):
    N = Zi.shape[0]
    A, Fd = emb.shape
    grid = N // _TN

    def body(z_ref, emb_ref, wi_ref, bi_ref, x_ref, v_ref):
        ar = lax.broadcasted_iota(jnp.int32, (_TN, A), 1)
        onehot = (ar == z_ref[...]).astype(jnp.float32)
        x = _dot(onehot, emb_ref[...])
        x_ref[...] = x
        v_ref[...] = _dot(x, wi_ref[...]) + bi_ref[...]

    return pl.pallas_call(
        body,
        grid=(grid,),
        in_specs=[
            pl.BlockSpec((_TN, 1), lambda i: (i, 0)),
            pl.BlockSpec((A, Fd), lambda i: (0, 0)),
            pl.BlockSpec((Fd, Fd), lambda i: (0, 0)),
            pl.BlockSpec((1, Fd), lambda i: (0, 0)),
        ],
        out_specs=[
            pl.BlockSpec((_TN, Fd), lambda i: (i, 0)),
            pl.BlockSpec((_TN, Fd), lambda i: (i, 0)),
        ],
        out_shape=[
            jax.ShapeDtypeStruct((N, Fd), jnp.float32),
            jax.ShapeDtypeStruct((N, Fd), jnp.float32),
        ],
    )(Zi, emb, w_in, b_in)


def _filter_conv(rbf_ref, vj_ref, x_ref, fw1, fb1, fw2, fb2, w1, b1, w2, b2,
                 K, Fd, R):
    rows = _TN * K
    u = _ssp(_bdot(rbf_ref[...].reshape(rows, R), fw1[...]) + fb1[...])
    w = _bdot(u, fw2[...]) + fb2[...]
    p = w * vj_ref[...]
    y = p.reshape(_TN, K, Fd).sum(axis=1)
    y = _ssp(_dot(y, w1[...]) + b1[...])
    y = _dot(y, w2[...]) + b2[...]
    return x_ref[...] + y


def _interaction_mid(x, vj, rbf3, blk, nxt, K):
    N, Fd = x.shape
    R = rbf3.shape[-1]
    grid = N // _TN
    rows = _TN * K

    def body(rbf_ref, vj_ref, x_ref, fw1, fb1, fw2, fb2, w1, b1, w2, b2,
             wi, bi, xo_ref, vn_ref):
        xo = _filter_conv(rbf_ref, vj_ref, x_ref, fw1, fb1, fw2, fb2,
                          w1, b1, w2, b2, K, Fd, R)
        xo_ref[...] = xo
        vn_ref[...] = _dot(xo, wi[...]) + bi[...]

    wspec = lambda s: pl.BlockSpec(s, lambda i: (0, 0))
    return pl.pallas_call(
        body,
        grid=(grid,),
        in_specs=[
            pl.BlockSpec((_TN, K, R), lambda i: (i, 0, 0)),
            pl.BlockSpec((rows, Fd), lambda i: (i, 0)),
            pl.BlockSpec((_TN, Fd), lambda i: (i, 0)),
            wspec((R, Fd)), wspec((1, Fd)), wspec((Fd, Fd)), wspec((1, Fd)),
            wspec((Fd, Fd)), wspec((1, Fd)), wspec((Fd, Fd)), wspec((1, Fd)),
            wspec((Fd, Fd)), wspec((1, Fd)),
        ],
        out_specs=[
            pl.BlockSpec((_TN, Fd), lambda i: (i, 0)),
            pl.BlockSpec((_TN, Fd), lambda i: (i, 0)),
        ],
        out_shape=[
            jax.ShapeDtypeStruct((N, Fd), jnp.float32),
            jax.ShapeDtypeStruct((N, Fd), jnp.float32),
        ],
    )(rbf3, vj, x, blk["fw1"], blk["fb1"].reshape(1, Fd),
      blk["fw2"], blk["fb2"].reshape(1, Fd),
      blk["w1"], blk["b1"].reshape(1, Fd),
      blk["w2"], blk["b2"].reshape(1, Fd),
      nxt["w_in"], nxt["b_in"].reshape(1, Fd))


def _interaction_last(x, vj, rbf3, blk, ro, K):
    N, Fd = x.shape
    R = rbf3.shape[-1]
    H = ro["rw1"].shape[1]
    grid = N // _TN
    rows = _TN * K

    def body(rbf_ref, vj_ref, x_ref, fw1, fb1, fw2, fb2, w1, b1, w2, b2,
             rw1, rb1, rw2, rb2, out_ref, acc_ref):
        xo = _filter_conv(rbf_ref, vj_ref, x_ref, fw1, fb1, fw2, fb2,
                          w1, b1, w2, b2, K, Fd, R)
        h = _ssp(_dot(xo, rw1[...]) + rb1[...])
        i = pl.program_id(0)

        @pl.when(i == 0)
        def _():
            acc_ref[...] = jnp.zeros_like(acc_ref)

        acc_ref[...] += jnp.sum(h, axis=0, keepdims=True)

        @pl.when(i == grid - 1)
        def _():
            out_ref[...] = _dot(acc_ref[...], rw2[...]) + N * rb2[...]

    wspec = lambda s: pl.BlockSpec(s, lambda i: (0, 0))
    return pl.pallas_call(
        body,
        grid=(grid,),
        in_specs=[
            pl.BlockSpec((_TN, K, R), lambda i: (i, 0, 0)),
            pl.BlockSpec((rows, Fd), lambda i: (i, 0)),
            pl.BlockSpec((_TN, Fd), lambda i: (i, 0)),
            wspec((R, Fd)), wspec((1, Fd)), wspec((Fd, Fd)), wspec((1, Fd)),
            wspec((Fd, Fd)), wspec((1, Fd)), wspec((Fd, Fd)), wspec((1, Fd)),
            wspec((Fd, H)), wspec((1, H)), wspec((H, 1)), wspec((1, 1)),
        ],
        out_specs=pl.BlockSpec((1, 1), lambda i: (0, 0)),
        out_shape=jax.ShapeDtypeStruct((1, 1), jnp.float32),
        scratch_shapes=[pltpu.VMEM((1, H), jnp.float32)],
        compiler_params=pltpu.CompilerParams(
            dimension_semantics=("arbitrary",)),
    )(rbf3, vj, x, blk["fw1"], blk["fb1"].reshape(1, Fd),
      blk["fw2"], blk["fb2"].reshape(1, Fd),
      blk["w1"], blk["b1"].reshape(1, Fd),
      blk["w2"], blk["b2"].reshape(1, Fd),
      ro["rw1"], ro["rb1"].reshape(1, H), ro["rw2"],
      ro["rb2"].reshape(1, 1))


def kernel(Z, rbf, neighbors, params):
    emb = params["embedding"]
    blocks = params["blocks"]
    ro = params["readout"]
    N, K = neighbors.shape
    R = rbf.shape[-1]
    Fd = emb.shape[1]
    T = len(blocks)

    nbr = neighbors.reshape(N * K).astype(jnp.int32)
    Zi = Z.astype(jnp.int32).reshape(N, 1)
    rbf_b = rbf.astype(jnp.bfloat16)

    x, v = _embed_call(Zi, emb, blocks[0]["w_in"],
                       blocks[0]["b_in"].reshape(1, Fd))
    for t in range(T - 1):
        vj = _sc_gather(v, nbr)
        x, v = _interaction_mid(x, vj, rbf_b, blocks[t], blocks[t + 1], K)
    vj = _sc_gather(v, nbr)
    e = _interaction_last(x, vj, rbf_b, blocks[T - 1], ro, K)
    return e.reshape(())
